# baseline (speedup 1.0000x reference)
import functools

import jax
import jax.numpy as jnp
from jax import lax
from jax.experimental import pallas as pl
from jax.experimental.pallas import tpu as pltpu
from jax.experimental.pallas import tpu_sc as plsc

_INFO = plsc.get_sparse_core_info()
_NC, _NS = _INFO.num_cores, _INFO.num_subcores
_NW = _NC * _NS
_N_CHUNKS = 25


def _make_edges_copy(n_edges, d_edge, dtype):
    rows_per_w = n_edges // _NW
    chunk = rows_per_w // _N_CHUNKS
    mesh = plsc.VectorSubcoreMesh(core_axis_name="c", subcore_axis_name="s")

    @functools.partial(
        pl.kernel,
        mesh=mesh,
        out_type=jax.ShapeDtypeStruct((n_edges, d_edge), dtype),
        scratch_types=[
            pltpu.VMEM((chunk, d_edge), dtype),
            pltpu.VMEM((chunk, d_edge), dtype),
            pltpu.SemaphoreType.DMA,
            pltpu.SemaphoreType.DMA,
            pltpu.SemaphoreType.DMA,
            pltpu.SemaphoreType.DMA,
        ],
    )
    def k(e_hbm, out_hbm, b0, b1, si0, si1, so0, so1):
        wid = lax.axis_index("s") * _NC + lax.axis_index("c")
        base = wid * rows_per_w
        bufs = (b0, b1)
        sin = (si0, si1)
        sout = (so0, so1)

        def src(c):
            return e_hbm.at[pl.ds(base + c * chunk, chunk)]

        def dst(c):
            return out_hbm.at[pl.ds(base + c * chunk, chunk)]

        ins = {}
        outs = {}
        ins[0] = pltpu.async_copy(src(0), bufs[0], sin[0])
        for c in range(_N_CHUNKS):
            b = c % 2
            ins[c].wait()
            if c + 1 < _N_CHUNKS:
                nb = (c + 1) % 2
                if c >= 1:
                    outs[c - 1].wait()
                ins[c + 1] = pltpu.async_copy(src(c + 1), bufs[nb], sin[nb])
            outs[c] = pltpu.async_copy(bufs[b], dst(c), sout[b])
        outs[_N_CHUNKS - 2].wait()
        outs[_N_CHUNKS - 1].wait()

    return k


def kernel(nodes, edge_index, edges=None, u=None, batch=None):
    if batch is None:
        batch = jnp.zeros((nodes.shape[0],), dtype=jnp.int32)
    n_edges, d_edge = edges.shape
    edges_o = _make_edges_copy(n_edges, d_edge, edges.dtype)(edges)
    return (nodes, edge_index, edges_o, u, batch)
